# baseline (speedup 1.0000x reference)
import functools
import math

import jax
import jax.numpy as jnp
from jax import lax
from jax.experimental import pallas as pl
from jax.experimental.pallas import tpu as pltpu
from jax.experimental.pallas import tpu_sc as plsc

D_MODEL = 64
SCALE = math.sqrt(D_MODEL)
NUM_CORES = 2
NUM_SUBCORES = 16
NW = NUM_CORES * NUM_SUBCORES
CHUNKS = (104, 96)
OFFS = (0, 104)
CMAX = 104
NBUF = 4


@functools.lru_cache(maxsize=None)
def _make_embed(nrows: int, seq: int, vocab: int):
    assert nrows % NW == 0 and seq == sum(CHUNKS)
    r_per_w = nrows // NW
    n_chunks = r_per_w * 2
    n_groups = n_chunks // NBUF
    mesh = plsc.VectorSubcoreMesh(core_axis_name="c", subcore_axis_name="s")

    @functools.partial(
        pl.kernel,
        mesh=mesh,
        compiler_params=pltpu.CompilerParams(use_tc_tiling_on_sc=False),
        out_type=jax.ShapeDtypeStruct((nrows, seq, D_MODEL), jnp.float32),
        scratch_types=[
            pltpu.VMEM((r_per_w, seq), jnp.int32),
            pltpu.VMEM((NBUF, CMAX, D_MODEL), jnp.float32),
            pltpu.VMEM((NBUF, CMAX, D_MODEL), jnp.float32),
            pltpu.SemaphoreType.DMA((NBUF,)),
            pltpu.SemaphoreType.DMA((NBUF,)),
        ],
    )
    def embed(x_hbm, lut_hbm, out_hbm, idx_v, gbuf, wbuf, gsem, wsem):
        wid = lax.axis_index("s") * NUM_CORES + lax.axis_index("c")
        row0 = wid * r_per_w
        pltpu.sync_copy(x_hbm.at[pl.ds(row0, r_per_w)], idx_v)

        def start_gather(r, h, b):
            pltpu.async_copy(
                lut_hbm.at[idx_v.at[r, pl.ds(OFFS[h], CHUNKS[h])]],
                gbuf.at[b, pl.ds(0, CHUNKS[h])],
                gsem.at[b],
            )

        for b in range(NBUF):
            start_gather(b // 2, b % 2, b)

        def group_body(cc, carry):
            for k in range(NBUF):
                h = k % 2
                ch = CHUNKS[h]
                r = cc * (NBUF // 2) + k // 2
                pltpu.make_async_copy(
                    lut_hbm.at[idx_v.at[0, pl.ds(0, ch)]],
                    gbuf.at[k, pl.ds(0, ch)],
                    gsem.at[k],
                ).wait()

                @pl.when(cc > 0)
                def _wait_wb():
                    pltpu.make_async_copy(
                        wbuf.at[k, pl.ds(0, ch)],
                        out_hbm.at[0, pl.ds(0, ch)],
                        wsem.at[k],
                    ).wait()

                def row_body(i, carry2):
                    for j in range(D_MODEL // 16):
                        sl = pl.ds(j * 16, 16)
                        wbuf[k, i, sl] = gbuf[k, i, sl] * SCALE
                    return carry2

                lax.fori_loop(0, ch, row_body, 0, unroll=4)

                pltpu.async_copy(
                    wbuf.at[k, pl.ds(0, ch)],
                    out_hbm.at[row0 + r, pl.ds(OFFS[h], ch)],
                    wsem.at[k],
                )

                @pl.when(cc < n_groups - 1)
                def _next_gather():
                    start_gather(r + NBUF // 2, h, k)

            return carry

        lax.fori_loop(0, n_groups, group_body, 0)

        for b in range(NBUF):
            ch = CHUNKS[b % 2]
            pltpu.make_async_copy(
                wbuf.at[b, pl.ds(0, ch)],
                out_hbm.at[0, pl.ds(0, ch)],
                wsem.at[b],
            ).wait()

    return embed


def kernel(x, lut):
    b, s = x.shape
    return _make_embed(b, s, lut.shape[0])(x.astype(jnp.int32), lut)
